# baseline (speedup 1.0000x reference)
import functools

import jax
import jax.numpy as jnp
from jax import lax
from jax.experimental import pallas as pl
from jax.experimental.pallas import tpu as pltpu
from jax.experimental.pallas import tpu_sc as plsc

_D = 32
_L = 20
_B = 4096
_N = 26
_LANES = 16
_NC = 2
_NS = 16
_NW = _NC * _NS
_BSLAB = _B // _NW
_DH = 8
_NQ = _D // _DH
_VPB = _BSLAB // _LANES


def _tree(vs):
    while len(vs) > 1:
        nxt = [vs[i] + vs[i + 1] for i in range(0, len(vs) - 1, 2)]
        if len(vs) % 2:
            nxt.append(vs[-1])
        vs = nxt
    return vs[0]


@functools.lru_cache(maxsize=None)
def _make_sc_call(nsc: int):
    mesh = plsc.VectorSubcoreMesh(core_axis_name="c", subcore_axis_name="s")

    def body(xt_hbm, mt_hbm, out_hbm,
             mbuf, inbuf0, inbuf1, outbuf0, outbuf1,
             msem, sem0, sem1, osem0, osem1):
        c = lax.axis_index("c")
        s = lax.axis_index("s")
        wid = s * _NC + c
        b0 = pl.multiple_of(wid * _BSLAB, _BSLAB)

        pltpu.async_copy(mt_hbm, mbuf, msem).wait()

        def compute(inbuf, outbuf, osem, n, d0, first):
            @pl.when(jnp.logical_not(first))
            def _():
                pltpu.make_async_copy(
                    outbuf,
                    out_hbm.at[0, pl.ds(d0, _DH), pl.ds(b0, _BSLAB)],
                    osem).wait()

            def db_body(db, carry):
                mbase = (d0 + db) * _L
                mreg = [mbuf[mbase + l, :] for l in range(_L)]
                for vi in range(_VPB):
                    p = [inbuf[l, db, pl.ds(vi * _LANES, _LANES)] * mreg[l]
                         for l in range(_L)]
                    outbuf[db, pl.ds(vi * _LANES, _LANES)] = _tree(p)
                return carry

            lax.fori_loop(0, _DH, db_body, 0)
            pltpu.async_copy(
                outbuf,
                out_hbm.at[n, pl.ds(d0, _DH), pl.ds(b0, _BSLAB)], osem)

        def start_in(n, d0, inbuf, sem):
            pltpu.async_copy(
                xt_hbm.at[n, :, pl.ds(d0, _DH), pl.ds(b0, _BSLAB)],
                inbuf, sem)

        def wait_in(n, d0, inbuf, sem):
            pltpu.make_async_copy(
                xt_hbm.at[n, :, pl.ds(d0, _DH), pl.ds(b0, _BSLAB)],
                inbuf, sem).wait()

        start_in(0, 0, inbuf0, sem0)

        bufs = [(inbuf0, outbuf0, sem0, osem0), (inbuf1, outbuf1, sem1, osem1)]

        def step(n, carry):
            for q in range(_NQ):
                ib, ob, isem, osem = bufs[q % 2]
                nib, _, nisem, _ = bufs[(q + 1) % 2]
                wait_in(n, q * _DH, ib, isem)
                if q < _NQ - 1:
                    start_in(n, (q + 1) * _DH, nib, nisem)
                else:
                    @pl.when(n < nsc - 1)
                    def _():
                        start_in(n + 1, 0, nib, nisem)
                compute(ib, ob, osem, n, q * _DH, jnp.logical_and(n == 0, q < 2))
            return carry

        lax.fori_loop(0, nsc, step, 0)
        pltpu.make_async_copy(
            outbuf0, out_hbm.at[0, pl.ds(0, _DH), pl.ds(b0, _BSLAB)],
            osem0).wait()
        pltpu.make_async_copy(
            outbuf1, out_hbm.at[0, pl.ds(_DH, _DH), pl.ds(b0, _BSLAB)],
            osem1).wait()

    return pl.kernel(
        body,
        out_type=jax.ShapeDtypeStruct((nsc, _D, _B), jnp.float32),
        mesh=mesh,
        scratch_types=[
            pltpu.VMEM((_D * _L, _LANES), jnp.float32),
            pltpu.VMEM((_L, _DH, _BSLAB), jnp.float32),
            pltpu.VMEM((_L, _DH, _BSLAB), jnp.float32),
            pltpu.VMEM((_DH, _BSLAB), jnp.float32),
            pltpu.VMEM((_DH, _BSLAB), jnp.float32),
            pltpu.SemaphoreType.DMA,
            pltpu.SemaphoreType.DMA,
            pltpu.SemaphoreType.DMA,
            pltpu.SemaphoreType.DMA,
            pltpu.SemaphoreType.DMA,
        ],
    )


_BB = 512
_NSC = 0


@functools.lru_cache(maxsize=None)
def _make_tc_call(nsc: int):
    ntc = _N - nsc

    def tc_body(x_ref, m_ref, o_ref):
        o_ref[0] = jnp.sum(x_ref[0] * m_ref[...], axis=0)

    return pl.pallas_call(
        tc_body,
        grid=(ntc, _B // _BB),
        in_specs=[
            pl.BlockSpec((1, _L, _D, _BB), lambda i, j: (i + nsc, 0, 0, j)),
            pl.BlockSpec((_L, _D, _BB), lambda i, j: (0, 0, 0)),
        ],
        out_specs=pl.BlockSpec((1, _D, _BB), lambda i, j: (i, 0, j)),
        out_shape=jax.ShapeDtypeStruct((ntc, _D, _B), jnp.float32),
    )


def kernel(x, mask_table):
    B, N, L, D = x.shape
    assert (B, N, L, D) == (_B, _N, _L, _D)
    xt = jnp.transpose(x, (1, 2, 3, 0))
    mb = jnp.broadcast_to(
        mask_table[:_L].T[:, :, None], (_D, _L, _LANES)).reshape(
            _D * _L, _LANES)
    mt = jnp.broadcast_to(mask_table[:_L][:, :, None], (_L, _D, _BB))
    parts = []
    if _NSC > 0:
        parts.append(_make_sc_call(_NSC)(xt, mb))
    if _NSC < _N:
        parts.append(_make_tc_call(_NSC)(xt, mt))
    out = parts[0] if len(parts) == 1 else jnp.concatenate(parts, axis=0)
    return jnp.transpose(out, (2, 0, 1))


# speedup vs baseline: 1.4454x; 1.4454x over previous
import functools

import jax
import jax.numpy as jnp
from jax import lax
from jax.experimental import pallas as pl
from jax.experimental.pallas import tpu as pltpu
from jax.experimental.pallas import tpu_sc as plsc

_D = 32
_L = 20
_B = 4096
_N = 26
_LANES = 16
_NC = 2
_NS = 16
_NW = _NC * _NS
_BSLAB = _B // _NW
_DH = 8
_NQ = _D // _DH
_VPB = _BSLAB // _LANES


def _tree(vs):
    while len(vs) > 1:
        nxt = [vs[i] + vs[i + 1] for i in range(0, len(vs) - 1, 2)]
        if len(vs) % 2:
            nxt.append(vs[-1])
        vs = nxt
    return vs[0]


@functools.lru_cache(maxsize=None)
def _make_sc_call(nsc: int):
    mesh = plsc.VectorSubcoreMesh(core_axis_name="c", subcore_axis_name="s")

    def body(xt_hbm, mt_hbm, out_hbm,
             mbuf, inbuf0, inbuf1, outbuf0, outbuf1,
             msem, sem0, sem1, osem0, osem1):
        c = lax.axis_index("c")
        s = lax.axis_index("s")
        wid = s * _NC + c
        b0 = pl.multiple_of(wid * _BSLAB, _BSLAB)

        pltpu.async_copy(mt_hbm, mbuf, msem).wait()

        def compute(inbuf, outbuf, osem, n, d0, first):
            @pl.when(jnp.logical_not(first))
            def _():
                pltpu.make_async_copy(
                    outbuf,
                    out_hbm.at[0, pl.ds(d0, _DH), pl.ds(b0, _BSLAB)],
                    osem).wait()

            def db_body(db, carry):
                mbase = (d0 + db) * _L
                mreg = [mbuf[mbase + l, :] for l in range(_L)]
                for vi in range(_VPB):
                    p = [inbuf[l, db, pl.ds(vi * _LANES, _LANES)] * mreg[l]
                         for l in range(_L)]
                    outbuf[db, pl.ds(vi * _LANES, _LANES)] = _tree(p)
                return carry

            lax.fori_loop(0, _DH, db_body, 0)
            pltpu.async_copy(
                outbuf,
                out_hbm.at[n, pl.ds(d0, _DH), pl.ds(b0, _BSLAB)], osem)

        def start_in(n, d0, inbuf, sem):
            pltpu.async_copy(
                xt_hbm.at[n, :, pl.ds(d0, _DH), pl.ds(b0, _BSLAB)],
                inbuf, sem)

        def wait_in(n, d0, inbuf, sem):
            pltpu.make_async_copy(
                xt_hbm.at[n, :, pl.ds(d0, _DH), pl.ds(b0, _BSLAB)],
                inbuf, sem).wait()

        start_in(0, 0, inbuf0, sem0)

        bufs = [(inbuf0, outbuf0, sem0, osem0), (inbuf1, outbuf1, sem1, osem1)]

        def step(n, carry):
            for q in range(_NQ):
                ib, ob, isem, osem = bufs[q % 2]
                nib, _, nisem, _ = bufs[(q + 1) % 2]
                wait_in(n, q * _DH, ib, isem)
                if q < _NQ - 1:
                    start_in(n, (q + 1) * _DH, nib, nisem)
                else:
                    @pl.when(n < nsc - 1)
                    def _():
                        start_in(n + 1, 0, nib, nisem)
                compute(ib, ob, osem, n, q * _DH, jnp.logical_and(n == 0, q < 2))
            return carry

        lax.fori_loop(0, nsc, step, 0)
        pltpu.make_async_copy(
            outbuf0, out_hbm.at[0, pl.ds(0, _DH), pl.ds(b0, _BSLAB)],
            osem0).wait()
        pltpu.make_async_copy(
            outbuf1, out_hbm.at[0, pl.ds(_DH, _DH), pl.ds(b0, _BSLAB)],
            osem1).wait()

    return pl.kernel(
        body,
        out_type=jax.ShapeDtypeStruct((nsc, _D, _B), jnp.float32),
        mesh=mesh,
        scratch_types=[
            pltpu.VMEM((_D * _L, _LANES), jnp.float32),
            pltpu.VMEM((_L, _DH, _BSLAB), jnp.float32),
            pltpu.VMEM((_L, _DH, _BSLAB), jnp.float32),
            pltpu.VMEM((_DH, _BSLAB), jnp.float32),
            pltpu.VMEM((_DH, _BSLAB), jnp.float32),
            pltpu.SemaphoreType.DMA,
            pltpu.SemaphoreType.DMA,
            pltpu.SemaphoreType.DMA,
            pltpu.SemaphoreType.DMA,
            pltpu.SemaphoreType.DMA,
        ],
    )


_BB = 1024
_NSC = 0


@functools.lru_cache(maxsize=None)
def _make_tc_call(nsc: int):
    ntc = _N - nsc

    def tc_body(x_ref, m_ref, o_ref):
        o_ref[0] = jnp.sum(x_ref[0] * m_ref[...], axis=0)

    return pl.pallas_call(
        tc_body,
        grid=(ntc, _B // _BB),
        in_specs=[
            pl.BlockSpec((1, _L, _D, _BB), lambda i, j: (i + nsc, 0, 0, j)),
            pl.BlockSpec((_L, _D, _BB), lambda i, j: (0, 0, 0)),
        ],
        out_specs=pl.BlockSpec((1, _D, _BB), lambda i, j: (i, 0, j)),
        out_shape=jax.ShapeDtypeStruct((ntc, _D, _B), jnp.float32),
    )


def kernel(x, mask_table):
    B, N, L, D = x.shape
    assert (B, N, L, D) == (_B, _N, _L, _D)
    xt = jnp.transpose(x, (1, 2, 3, 0))
    mb = jnp.broadcast_to(
        mask_table[:_L].T[:, :, None], (_D, _L, _LANES)).reshape(
            _D * _L, _LANES)
    mt = jnp.broadcast_to(mask_table[:_L][:, :, None], (_L, _D, _BB))
    parts = []
    if _NSC > 0:
        parts.append(_make_sc_call(_NSC)(xt, mb))
    if _NSC < _N:
        parts.append(_make_tc_call(_NSC)(xt, mt))
    out = parts[0] if len(parts) == 1 else jnp.concatenate(parts, axis=0)
    return jnp.transpose(out, (2, 0, 1))


# speedup vs baseline: 1.8307x; 1.2666x over previous
import functools

import jax
import jax.numpy as jnp
from jax import lax
from jax.experimental import pallas as pl
from jax.experimental.pallas import tpu as pltpu
from jax.experimental.pallas import tpu_sc as plsc

_D = 32
_L = 20
_B = 4096
_N = 26
_LANES = 16
_NC = 2
_NS = 16
_NW = _NC * _NS
_BSLAB = _B // _NW
_DH = 8
_NQ = _D // _DH
_VPB = _BSLAB // _LANES


def _tree(vs):
    while len(vs) > 1:
        nxt = [vs[i] + vs[i + 1] for i in range(0, len(vs) - 1, 2)]
        if len(vs) % 2:
            nxt.append(vs[-1])
        vs = nxt
    return vs[0]


@functools.lru_cache(maxsize=None)
def _make_sc_call(nsc: int):
    mesh = plsc.VectorSubcoreMesh(core_axis_name="c", subcore_axis_name="s")

    def body(xt_hbm, mt_hbm, out_hbm,
             mbuf, inbuf0, inbuf1, outbuf0, outbuf1,
             msem, sem0, sem1, osem0, osem1):
        c = lax.axis_index("c")
        s = lax.axis_index("s")
        wid = s * _NC + c
        b0 = pl.multiple_of(wid * _BSLAB, _BSLAB)

        pltpu.async_copy(mt_hbm, mbuf, msem).wait()

        def compute(inbuf, outbuf, osem, n, d0, first):
            @pl.when(jnp.logical_not(first))
            def _():
                pltpu.make_async_copy(
                    outbuf,
                    out_hbm.at[0, pl.ds(d0, _DH), pl.ds(b0, _BSLAB)],
                    osem).wait()

            def db_body(db, carry):
                mbase = (d0 + db) * _L
                mreg = [mbuf[mbase + l, :] for l in range(_L)]
                for vi in range(_VPB):
                    p = [inbuf[l, db, pl.ds(vi * _LANES, _LANES)] * mreg[l]
                         for l in range(_L)]
                    outbuf[db, pl.ds(vi * _LANES, _LANES)] = _tree(p)
                return carry

            lax.fori_loop(0, _DH, db_body, 0)
            pltpu.async_copy(
                outbuf,
                out_hbm.at[n, pl.ds(d0, _DH), pl.ds(b0, _BSLAB)], osem)

        def start_in(n, d0, inbuf, sem):
            pltpu.async_copy(
                xt_hbm.at[n, :, pl.ds(d0, _DH), pl.ds(b0, _BSLAB)],
                inbuf, sem)

        def wait_in(n, d0, inbuf, sem):
            pltpu.make_async_copy(
                xt_hbm.at[n, :, pl.ds(d0, _DH), pl.ds(b0, _BSLAB)],
                inbuf, sem).wait()

        start_in(0, 0, inbuf0, sem0)

        bufs = [(inbuf0, outbuf0, sem0, osem0), (inbuf1, outbuf1, sem1, osem1)]

        def step(n, carry):
            for q in range(_NQ):
                ib, ob, isem, osem = bufs[q % 2]
                nib, _, nisem, _ = bufs[(q + 1) % 2]
                wait_in(n, q * _DH, ib, isem)
                if q < _NQ - 1:
                    start_in(n, (q + 1) * _DH, nib, nisem)
                else:
                    @pl.when(n < nsc - 1)
                    def _():
                        start_in(n + 1, 0, nib, nisem)
                compute(ib, ob, osem, n, q * _DH, jnp.logical_and(n == 0, q < 2))
            return carry

        lax.fori_loop(0, nsc, step, 0)
        pltpu.make_async_copy(
            outbuf0, out_hbm.at[0, pl.ds(0, _DH), pl.ds(b0, _BSLAB)],
            osem0).wait()
        pltpu.make_async_copy(
            outbuf1, out_hbm.at[0, pl.ds(_DH, _DH), pl.ds(b0, _BSLAB)],
            osem1).wait()

    return pl.kernel(
        body,
        out_type=jax.ShapeDtypeStruct((nsc, _D, _B), jnp.float32),
        mesh=mesh,
        scratch_types=[
            pltpu.VMEM((_D * _L, _LANES), jnp.float32),
            pltpu.VMEM((_L, _DH, _BSLAB), jnp.float32),
            pltpu.VMEM((_L, _DH, _BSLAB), jnp.float32),
            pltpu.VMEM((_DH, _BSLAB), jnp.float32),
            pltpu.VMEM((_DH, _BSLAB), jnp.float32),
            pltpu.SemaphoreType.DMA,
            pltpu.SemaphoreType.DMA,
            pltpu.SemaphoreType.DMA,
            pltpu.SemaphoreType.DMA,
            pltpu.SemaphoreType.DMA,
        ],
    )


_BB = 2048
_NSC = 0


@functools.lru_cache(maxsize=None)
def _make_tc_call(nsc: int):
    ntc = _N - nsc

    def tc_body(x_ref, m_ref, o_ref):
        o_ref[0] = jnp.sum(x_ref[0] * m_ref[...], axis=0)

    return pl.pallas_call(
        tc_body,
        grid=(ntc, _B // _BB),
        in_specs=[
            pl.BlockSpec((1, _L, _D, _BB), lambda i, j: (i + nsc, 0, 0, j)),
            pl.BlockSpec((_L, _D, _BB), lambda i, j: (0, 0, 0)),
        ],
        out_specs=pl.BlockSpec((1, _D, _BB), lambda i, j: (i, 0, j)),
        out_shape=jax.ShapeDtypeStruct((ntc, _D, _B), jnp.float32),
    )


def kernel(x, mask_table):
    B, N, L, D = x.shape
    assert (B, N, L, D) == (_B, _N, _L, _D)
    xt = jnp.transpose(x, (1, 2, 3, 0))
    mb = jnp.broadcast_to(
        mask_table[:_L].T[:, :, None], (_D, _L, _LANES)).reshape(
            _D * _L, _LANES)
    mt = jnp.broadcast_to(mask_table[:_L][:, :, None], (_L, _D, _BB))
    parts = []
    if _NSC > 0:
        parts.append(_make_sc_call(_NSC)(xt, mb))
    if _NSC < _N:
        parts.append(_make_tc_call(_NSC)(xt, mt))
    out = parts[0] if len(parts) == 1 else jnp.concatenate(parts, axis=0)
    return jnp.transpose(out, (2, 0, 1))
